# baseline (speedup 1.0000x reference)
import functools

import jax
import jax.numpy as jnp
from jax import lax
from jax.experimental import pallas as pl
from jax.experimental.pallas import tpu as pltpu
from jax.experimental.pallas import tpu_sc as plsc

_D = 64
_B = 4096
_S = 200
_NC = 2
_NS = 16
_NW = _NC * _NS
_NSPLIT = 2
_BS = _B // _NSPLIT
_ROWS_W = _BS // _NW
_CR = 4
_SPLITS = ((0, 104), (104, 96))
_NCHUNK = _ROWS_W // _CR
_NBUF = 2


def _gather_body(x_hbm, table_hbm, prev_hbm, out_hbm, idx_v, rows_v, gsem, ssem,
                 *, base):
    del prev_hbm
    cid = lax.axis_index("c")
    sid = lax.axis_index("s")
    wid = sid * _NC + cid
    row0 = wid * _ROWS_W

    def fire(chunk, b):
        r = base + row0 + chunk * _CR
        pltpu.sync_copy(x_hbm.at[pl.ds(r, _CR)], idx_v.at[b])
        for rr in range(_CR):
            for s0, w in _SPLITS:
                pltpu.async_copy(
                    table_hbm.at[idx_v.at[b, rr, pl.ds(s0, w)]],
                    rows_v.at[b, rr, pl.ds(s0, w)],
                    gsem.at[b],
                )

    def drain_and_store(chunk, b):
        for rr in range(_CR):
            for s0, w in _SPLITS:
                pltpu.make_async_copy(
                    table_hbm.at[idx_v.at[b, rr, pl.ds(s0, w)]],
                    rows_v.at[b, rr, pl.ds(s0, w)],
                    gsem.at[b],
                ).wait()
        r = row0 + chunk * _CR
        pltpu.async_copy(rows_v.at[b], out_hbm.at[pl.ds(r, _CR)], ssem.at[b])

    def wait_store(b):
        pltpu.make_async_copy(
            rows_v.at[b], out_hbm.at[pl.ds(row0, _CR)], ssem.at[b]
        ).wait()

    for b in range(_NBUF):
        fire(b, b)

    def round_body(i, carry):
        g0 = i * _NBUF
        for b in range(_NBUF):
            drain_and_store(g0 + b, b)
            nxt = g0 + b + _NBUF
            @pl.when(nxt < _NCHUNK)
            def _():
                wait_store(b)
                fire(nxt, b)
        return carry

    lax.fori_loop(0, _NCHUNK // _NBUF, round_body, 0)
    for b in range(_NBUF):
        wait_store(b)


@jax.jit
def kernel(x, table):
    xi = x if x.dtype == jnp.int32 else x.astype(jnp.int32)
    mesh = plsc.VectorSubcoreMesh(
        core_axis_name="c", subcore_axis_name="s", num_cores=_NC
    )
    outs = []
    for sp in range(_NSPLIT):
        gather = functools.partial(
            pl.kernel,
            mesh=mesh,
            out_type=jax.ShapeDtypeStruct((_BS, _S, _D), jnp.float32),
            scratch_types=[
                pltpu.VMEM((_NBUF, _CR, _S), jnp.int32),
                pltpu.VMEM((_NBUF, _CR, _S, _D), jnp.float32),
                pltpu.SemaphoreType.DMA((_NBUF,)),
                pltpu.SemaphoreType.DMA((_NBUF,)),
            ],
            compiler_params=pltpu.CompilerParams(use_tc_tiling_on_sc=False),
        )(functools.partial(_gather_body, base=sp * _BS))
        prev = xi if sp == 0 else outs[-1]
        outs.append(gather(xi, table, prev))
    return jnp.concatenate(outs, axis=0)


# speedup vs baseline: 1.1866x; 1.1866x over previous
import functools

import jax
import jax.numpy as jnp
from jax import lax
from jax.experimental import pallas as pl
from jax.experimental.pallas import tpu as pltpu
from jax.experimental.pallas import tpu_sc as plsc

_D = 64
_B = 4096
_S = 200
_NC = 2
_NS = 16
_NW = _NC * _NS
_ROWS_W = _B // _NW
_CR = 4
_SPLITS = ((0, 104), (104, 96))
_NCHUNK = _ROWS_W // _CR
_NBUF = 2


def _gather_body(x_hbm, table_hbm, out_hbm, idx_v, rows_v, gsem, ssem):
    cid = lax.axis_index("c")
    sid = lax.axis_index("s")
    wid = sid * _NC + cid
    row0 = wid * _ROWS_W

    def fire(chunk, b):
        r = row0 + chunk * _CR
        pltpu.sync_copy(x_hbm.at[pl.ds(r, _CR)], idx_v.at[b])
        for rr in range(_CR):
            for s0, w in _SPLITS:
                pltpu.async_copy(
                    table_hbm.at[idx_v.at[b, rr, pl.ds(s0, w)]],
                    rows_v.at[b, rr, pl.ds(s0, w)],
                    gsem.at[b],
                )

    def drain_and_store(chunk, b):
        for rr in range(_CR):
            for s0, w in _SPLITS:
                pltpu.make_async_copy(
                    table_hbm.at[idx_v.at[b, rr, pl.ds(s0, w)]],
                    rows_v.at[b, rr, pl.ds(s0, w)],
                    gsem.at[b],
                ).wait()
        r = row0 + chunk * _CR
        pltpu.async_copy(rows_v.at[b], out_hbm.at[pl.ds(r, _CR)], ssem.at[b])

    def wait_store(b):
        pltpu.make_async_copy(
            rows_v.at[b], out_hbm.at[pl.ds(row0, _CR)], ssem.at[b]
        ).wait()

    for b in range(_NBUF):
        fire(b, b)

    def round_body(i, carry):
        g0 = i * _NBUF
        for b in range(_NBUF):
            drain_and_store(g0 + b, b)
            nxt = g0 + b + _NBUF
            @pl.when(nxt < _NCHUNK)
            def _():
                wait_store(b)
                fire(nxt, b)
        return carry

    lax.fori_loop(0, _NCHUNK // _NBUF, round_body, 0)
    for b in range(_NBUF):
        wait_store(b)


@jax.jit
def kernel(x, table):
    xi = x if x.dtype == jnp.int32 else x.astype(jnp.int32)
    mesh = plsc.VectorSubcoreMesh(
        core_axis_name="c", subcore_axis_name="s", num_cores=_NC
    )
    gather = functools.partial(
        pl.kernel,
        mesh=mesh,
        out_type=jax.ShapeDtypeStruct((_B, _S, _D), jnp.float32),
        scratch_types=[
            pltpu.VMEM((_NBUF, _CR, _S), jnp.int32),
            pltpu.VMEM((_NBUF, _CR, _S, _D), jnp.float32),
            pltpu.SemaphoreType.DMA((_NBUF,)),
            pltpu.SemaphoreType.DMA((_NBUF,)),
        ],
        compiler_params=pltpu.CompilerParams(use_tc_tiling_on_sc=False),
    )(_gather_body)
    return gather(xi, table)


# speedup vs baseline: 1.5769x; 1.3290x over previous
import functools

import jax
import jax.numpy as jnp
from jax import lax
from jax.experimental import pallas as pl
from jax.experimental.pallas import tpu as pltpu
from jax.experimental.pallas import tpu_sc as plsc

_D = 64
_B = 4096
_S = 200
_NC = 2
_NS = 16
_NW = _NC * _NS
_ROWS_W = _B // _NW
_CR = 4
_SPLITS = ((0, 104), (104, 96))
_NCHUNK = _ROWS_W // _CR
_NBUF = 2


def _gather_body(x_hbm, table_hbm, out_hbm, idx_v, rows_v, gsem, ssem):
    cid = lax.axis_index("c")
    sid = lax.axis_index("s")
    wid = sid * _NC + cid
    row0 = wid * _ROWS_W

    def fire(chunk, b):
        r = row0 + chunk * _CR
        pltpu.sync_copy(x_hbm.at[pl.ds(r, _CR)], idx_v.at[b])
        for rr in range(_CR):
            for s0, w in _SPLITS:
                pltpu.async_copy(
                    table_hbm.at[idx_v.at[b, rr, pl.ds(s0, w)]],
                    rows_v.at[b, rr, pl.ds(s0, w)],
                    gsem.at[b],
                )

    def drain_and_store(chunk, b):
        for rr in range(_CR):
            for s0, w in _SPLITS:
                pltpu.make_async_copy(
                    table_hbm.at[idx_v.at[b, rr, pl.ds(s0, w)]],
                    rows_v.at[b, rr, pl.ds(s0, w)],
                    gsem.at[b],
                ).wait()
        r = row0 + chunk * _CR
        pltpu.async_copy(
            rows_v.at[b],
            out_hbm.at[pl.ds(r, _CR), :, pl.ds(0, _D)],
            ssem.at[b],
        )

    def wait_store(b):
        pltpu.make_async_copy(
            rows_v.at[b],
            out_hbm.at[pl.ds(row0, _CR), :, pl.ds(0, _D)],
            ssem.at[b],
        ).wait()

    for b in range(_NBUF):
        fire(b, b)

    def round_body(i, carry):
        g0 = i * _NBUF
        for b in range(_NBUF):
            drain_and_store(g0 + b, b)
            nxt = g0 + b + _NBUF
            @pl.when(nxt < _NCHUNK)
            def _():
                wait_store(b)
                fire(nxt, b)
        return carry

    lax.fori_loop(0, _NCHUNK // _NBUF, round_body, 0)
    for b in range(_NBUF):
        wait_store(b)


@jax.jit
def kernel(x, table):
    xi = x if x.dtype == jnp.int32 else x.astype(jnp.int32)
    mesh = plsc.VectorSubcoreMesh(
        core_axis_name="c", subcore_axis_name="s", num_cores=_NC
    )
    gather = functools.partial(
        pl.kernel,
        mesh=mesh,
        out_type=jax.ShapeDtypeStruct((_B, _S, 2 * _D), jnp.float32),
        scratch_types=[
            pltpu.VMEM((_NBUF, _CR, _S), jnp.int32),
            pltpu.VMEM((_NBUF, _CR, _S, _D), jnp.float32),
            pltpu.SemaphoreType.DMA((_NBUF,)),
            pltpu.SemaphoreType.DMA((_NBUF,)),
        ],
        compiler_params=pltpu.CompilerParams(use_tc_tiling_on_sc=False),
    )(_gather_body)
    return gather(xi, table)[:, :, :_D]
